# Initial kernel scaffold; baseline (speedup 1.0000x reference)
import jax
import jax.numpy as jnp
from jax.experimental import pallas as pl


def kernel(x, edge_index, W1, b1, W2, b2, W3, b3, noise):
    raise NotImplementedError("write your pallas kernel here")


# speedup vs baseline: 71.1691x; 71.1691x over previous
import functools

import jax
import jax.numpy as jnp
from jax import lax
from jax.experimental import pallas as pl
from jax.experimental.pallas import tpu as pltpu
from jax.experimental.pallas import tpu_sc as plsc

N = 10000
E = 320000
D = 128

NC = 2
NS = 16
NW = NC * NS
L = 16

E_PER = E // NW
N_CHUNKS = N // L
N_WORKERS_NODE = 25
N_PER = (N_CHUNKS // N_WORKERS_NODE) * L


def _mlp_body(x_ref, w1_ref, b1_ref, w2_ref, b2_ref, w3_ref, b3_ref,
              noise_ref, att_ref):
    eps = 1e-5
    h = jnp.dot(x_ref[...], w1_ref[...], preferred_element_type=jnp.float32)
    h = h + b1_ref[...]
    m = jnp.mean(h, axis=0, keepdims=True)
    c = h - m
    v = jnp.mean(c * c, axis=0, keepdims=True)
    h = jnp.maximum(c * lax.rsqrt(v + eps), 0.0)
    h = jnp.dot(h, w2_ref[...], preferred_element_type=jnp.float32)
    h = h + b2_ref[...]
    m = jnp.mean(h, axis=0, keepdims=True)
    c = h - m
    v = jnp.mean(c * c, axis=0, keepdims=True)
    h = jnp.maximum(c * lax.rsqrt(v + eps), 0.0)
    logit = jnp.dot(h, w3_ref[...], preferred_element_type=jnp.float32)
    logit = logit + b3_ref[...]
    nz = noise_ref[...]
    rn = jnp.log(nz) - jnp.log(1.0 - nz)
    att_ref[...] = jax.nn.sigmoid(logit + rn)


_mlp = pl.pallas_call(
    _mlp_body,
    out_shape=jax.ShapeDtypeStruct((N, 1), jnp.float32),
)


@functools.cache
def _build_lift():
    mesh = plsc.VectorSubcoreMesh(core_axis_name="c", subcore_axis_name="s")

    @functools.partial(
        pl.kernel,
        mesh=mesh,
        out_type=jax.ShapeDtypeStruct((E + N,), jnp.float32),
        scratch_types=[
            pltpu.VMEM((N,), jnp.float32),
            pltpu.VMEM((E_PER,), jnp.int32),
            pltpu.VMEM((E_PER,), jnp.int32),
            pltpu.VMEM((E_PER,), jnp.float32),
            pltpu.VMEM((N_PER,), jnp.float32),
        ],
        compiler_params=pltpu.CompilerParams(needs_layout_passes=False),
    )
    def _lift(att_hbm, eidx_hbm, out_hbm, att_v, src_v, dst_v, eo_v, no_v):
        wid = lax.axis_index("s") * NC + lax.axis_index("c")
        ebase = wid * E_PER
        pltpu.sync_copy(att_hbm, att_v)
        pltpu.sync_copy(eidx_hbm.at[pl.ds(ebase, E_PER)], src_v)
        pltpu.sync_copy(eidx_hbm.at[pl.ds(E + ebase, E_PER)], dst_v)

        def edge_body(i, carry):
            s = plsc.load_gather(att_v, [src_v[pl.ds(i * L, L)]])
            d = plsc.load_gather(att_v, [dst_v[pl.ds(i * L, L)]])
            eo_v[pl.ds(i * L, L)] = s * d
            return carry

        lax.fori_loop(0, E_PER // L, edge_body, 0)
        pltpu.sync_copy(eo_v, out_hbm.at[pl.ds(ebase, E_PER)])

        @pl.when(wid < N_WORKERS_NODE)
        def _node_part():
            nbase = wid * N_PER

            def node_body(i, carry):
                a = att_v[pl.ds(nbase + i * L, L)]
                no_v[pl.ds(i * L, L)] = a * a
                return carry

            lax.fori_loop(0, N_PER // L, node_body, 0)
            pltpu.sync_copy(no_v, out_hbm.at[pl.ds(E + nbase, N_PER)])

    return _lift


def kernel(x, edge_index, W1, b1, W2, b2, W3, b3, noise):
    att = _mlp(x, W1, b1.reshape(1, 2 * D), W2, b2.reshape(1, D),
               W3, b3.reshape(1, 1), noise)
    out = _build_lift()(att.reshape(N), edge_index.reshape(2 * E))
    return out.reshape(E + N, 1)


# speedup vs baseline: 84.8328x; 1.1920x over previous
import functools

import jax
import jax.numpy as jnp
from jax import lax
from jax.experimental import pallas as pl
from jax.experimental.pallas import tpu as pltpu
from jax.experimental.pallas import tpu_sc as plsc

N = 10000
E = 320000
D = 128

NC = 2
NS = 16
NW = NC * NS
L = 16

E_PER = E // NW
N_CHUNKS = N // L
N_WORKERS_NODE = 25
N_PER = (N_CHUNKS // N_WORKERS_NODE) * L


def _mlp_body(x_ref, w1_ref, b1_ref, w2_ref, b2_ref, w3_ref, b3_ref,
              rn_ref, att_ref):
    eps = 1e-5
    h = lax.dot_general(w1_ref[...], x_ref[...], (((0,), (1,)), ((), ())),
                        preferred_element_type=jnp.float32)
    h = h + b1_ref[...]
    m = jnp.mean(h, axis=1, keepdims=True)
    c = h - m
    v = jnp.mean(c * c, axis=1, keepdims=True)
    h = jnp.maximum(c * lax.rsqrt(v + eps), 0.0)
    h = lax.dot_general(w2_ref[...], h, (((0,), (0,)), ((), ())),
                        preferred_element_type=jnp.float32)
    h = h + b2_ref[...]
    m = jnp.mean(h, axis=1, keepdims=True)
    c = h - m
    v = jnp.mean(c * c, axis=1, keepdims=True)
    h = jnp.maximum(c * lax.rsqrt(v + eps), 0.0)
    logit = lax.dot_general(w3_ref[...], h, (((0,), (0,)), ((), ())),
                            preferred_element_type=jnp.float32)
    logit = logit + b3_ref[...]
    z = logit + rn_ref[...].reshape(1, N)
    att_ref[...] = jax.nn.sigmoid(z).reshape(N)


_mlp = pl.pallas_call(
    _mlp_body,
    out_shape=jax.ShapeDtypeStruct((N,), jnp.float32),
)


@functools.cache
def _build_lift():
    mesh = plsc.VectorSubcoreMesh(core_axis_name="c", subcore_axis_name="s")

    @functools.partial(
        pl.kernel,
        mesh=mesh,
        out_type=jax.ShapeDtypeStruct((E + N,), jnp.float32),
        scratch_types=[
            pltpu.VMEM((N,), jnp.float32),
            pltpu.VMEM((E_PER,), jnp.int32),
            pltpu.VMEM((E_PER,), jnp.int32),
            pltpu.VMEM((E_PER,), jnp.float32),
            pltpu.VMEM((N_PER,), jnp.float32),
        ],
        compiler_params=pltpu.CompilerParams(needs_layout_passes=False),
    )
    def _lift(att_hbm, eidx_hbm, out_hbm, att_v, src_v, dst_v, eo_v, no_v):
        wid = lax.axis_index("s") * NC + lax.axis_index("c")
        ebase = wid * E_PER
        pltpu.sync_copy(att_hbm, att_v)
        pltpu.sync_copy(eidx_hbm.at[pl.ds(ebase, E_PER)], src_v)
        pltpu.sync_copy(eidx_hbm.at[pl.ds(E + ebase, E_PER)], dst_v)

        def edge_body(i, carry):
            s = plsc.load_gather(att_v, [src_v[pl.ds(i * L, L)]])
            d = plsc.load_gather(att_v, [dst_v[pl.ds(i * L, L)]])
            eo_v[pl.ds(i * L, L)] = s * d
            return carry

        lax.fori_loop(0, E_PER // L, edge_body, 0)
        pltpu.sync_copy(eo_v, out_hbm.at[pl.ds(ebase, E_PER)])

        @pl.when(wid < N_WORKERS_NODE)
        def _node_part():
            nbase = wid * N_PER

            def node_body(i, carry):
                a = att_v[pl.ds(nbase + i * L, L)]
                no_v[pl.ds(i * L, L)] = a * a
                return carry

            lax.fori_loop(0, N_PER // L, node_body, 0)
            pltpu.sync_copy(no_v, out_hbm.at[pl.ds(E + nbase, N_PER)])

    return _lift


def kernel(x, edge_index, W1, b1, W2, b2, W3, b3, noise):
    rn = (jnp.log(noise) - jnp.log(1.0 - noise)).reshape(N)
    att = _mlp(x, W1, b1.reshape(2 * D, 1), W2, b2.reshape(D, 1),
               W3, b3.reshape(1, 1), rn)
    out = _build_lift()(att, edge_index.reshape(2 * E))
    return out.reshape(E + N, 1)


# speedup vs baseline: 92.7670x; 1.0935x over previous
import functools

import jax
import jax.numpy as jnp
from jax import lax
from jax.experimental import pallas as pl
from jax.experimental.pallas import tpu as pltpu
from jax.experimental.pallas import tpu_sc as plsc

N = 10000
E = 320000
D = 128

NC = 2
NS = 16
NW = NC * NS
L = 16

E_PER = E // NW
N_CHUNKS = N // L
N_WORKERS_NODE = 25
N_PER = (N_CHUNKS // N_WORKERS_NODE) * L


def _mlp_body(x_ref, w1_ref, w2_ref, w3_ref, b3_ref, rn_ref, att_ref):
    eps = 1e-5
    h = lax.dot_general(w1_ref[...], x_ref[...], (((0,), (1,)), ((), ())),
                        preferred_element_type=jnp.float32)
    m = jnp.mean(h, axis=1, keepdims=True)
    c = h - m
    v = jnp.mean(c * c, axis=1, keepdims=True)
    h = jnp.maximum(c * lax.rsqrt(v + eps), 0.0)
    h = lax.dot_general(w2_ref[...], h, (((0,), (0,)), ((), ())),
                        preferred_element_type=jnp.float32)
    m = jnp.mean(h, axis=1, keepdims=True)
    c = h - m
    v = jnp.mean(c * c, axis=1, keepdims=True)
    h = jnp.maximum(c * lax.rsqrt(v + eps), 0.0)
    logit = lax.dot_general(w3_ref[...], h, (((1,), (0,)), ((), ())),
                            preferred_element_type=jnp.float32)
    logit = logit + b3_ref[...]
    z = logit + rn_ref[...].reshape(1, N)
    att_ref[...] = jax.nn.sigmoid(z).reshape(N)


_mlp = pl.pallas_call(
    _mlp_body,
    out_shape=jax.ShapeDtypeStruct((N,), jnp.float32),
)


@functools.cache
def _build_lift():
    mesh = plsc.VectorSubcoreMesh(core_axis_name="c", subcore_axis_name="s")

    @functools.partial(
        pl.kernel,
        mesh=mesh,
        out_type=jax.ShapeDtypeStruct((E + N,), jnp.float32),
        scratch_types=[
            pltpu.VMEM((N,), jnp.float32),
            pltpu.VMEM((E_PER,), jnp.int32),
            pltpu.VMEM((E_PER,), jnp.int32),
            pltpu.VMEM((E_PER,), jnp.float32),
            pltpu.VMEM((N_PER,), jnp.float32),
        ],
        compiler_params=pltpu.CompilerParams(needs_layout_passes=False),
    )
    def _lift(att_hbm, eidx_hbm, out_hbm, att_v, src_v, dst_v, eo_v, no_v):
        wid = lax.axis_index("s") * NC + lax.axis_index("c")
        ebase = wid * E_PER
        pltpu.sync_copy(att_hbm, att_v)
        pltpu.sync_copy(eidx_hbm.at[pl.ds(ebase, E_PER)], src_v)
        pltpu.sync_copy(eidx_hbm.at[pl.ds(E + ebase, E_PER)], dst_v)

        def edge_body(i, carry):
            s = plsc.load_gather(att_v, [src_v[pl.ds(i * L, L)]])
            d = plsc.load_gather(att_v, [dst_v[pl.ds(i * L, L)]])
            eo_v[pl.ds(i * L, L)] = s * d
            return carry

        lax.fori_loop(0, E_PER // L, edge_body, 0)
        pltpu.sync_copy(eo_v, out_hbm.at[pl.ds(ebase, E_PER)])

        @pl.when(wid < N_WORKERS_NODE)
        def _node_part():
            nbase = wid * N_PER

            def node_body(i, carry):
                a = att_v[pl.ds(nbase + i * L, L)]
                no_v[pl.ds(i * L, L)] = a * a
                return carry

            lax.fori_loop(0, N_PER // L, node_body, 0)
            pltpu.sync_copy(no_v, out_hbm.at[pl.ds(E + nbase, N_PER)])

    return _lift


def kernel(x, edge_index, W1, b1, W2, b2, W3, b3, noise):
    rn = (jnp.log(noise) - jnp.log(1.0 - noise)).reshape(N)
    att = _mlp(x, W1, W2, W3.reshape(1, D), b3.reshape(1, 1), rn)
    out = _build_lift()(att, edge_index.reshape(2 * E))
    return out.reshape(E + N, 1)


# speedup vs baseline: 110.0712x; 1.1865x over previous
import functools

import jax
import jax.numpy as jnp
from jax import lax
from jax.experimental import pallas as pl
from jax.experimental.pallas import tpu as pltpu
from jax.experimental.pallas import tpu_sc as plsc

N = 10000
E = 320000
D = 128

NC = 2
NS = 16
NW = NC * NS
L = 16

CT = E // 128
T_SML = CT // NW
T_BIG = T_SML + 1
N_BIG = CT - NW * T_SML
SZ_BIG = T_BIG * 128
SZ_SML = T_SML * 128

N_CHUNKS = N // L
N_WORKERS_NODE = 25
N_PER = (N_CHUNKS // N_WORKERS_NODE) * L


def _mlp_body(x_ref, w1_ref, w2_ref, w3_ref, b3_ref, rn_ref, att_ref):
    eps = 1e-5
    h = lax.dot_general(w1_ref[...], x_ref[...], (((0,), (1,)), ((), ())),
                        preferred_element_type=jnp.float32)
    m = jnp.mean(h, axis=1, keepdims=True)
    c = h - m
    v = jnp.mean(c * c, axis=1, keepdims=True)
    h = jnp.maximum(c * lax.rsqrt(v + eps), 0.0)
    h = lax.dot_general(w2_ref[...], h, (((0,), (0,)), ((), ())),
                        preferred_element_type=jnp.float32)
    m = jnp.mean(h, axis=1, keepdims=True)
    c = h - m
    v = jnp.mean(c * c, axis=1, keepdims=True)
    h = jnp.maximum(c * lax.rsqrt(v + eps), 0.0)
    logit = lax.dot_general(w3_ref[...], h, (((1,), (0,)), ((), ())),
                            preferred_element_type=jnp.float32)
    logit = logit + b3_ref[...]
    z = logit + rn_ref[...].reshape(1, N)
    att_ref[...] = jax.nn.sigmoid(z).reshape(N)


_mlp = pl.pallas_call(
    _mlp_body,
    out_shape=jax.ShapeDtypeStruct((N,), jnp.float32),
)


@functools.cache
def _build_lift():
    mesh = plsc.VectorSubcoreMesh(core_axis_name="c", subcore_axis_name="s")

    @functools.partial(
        pl.kernel,
        mesh=mesh,
        out_type=jax.ShapeDtypeStruct((E + N,), jnp.float32),
        scratch_types=[
            pltpu.VMEM((N,), jnp.float32),
            pltpu.VMEM((2, SZ_BIG), jnp.int32),
            pltpu.VMEM((SZ_BIG,), jnp.float32),
            pltpu.VMEM((N_PER,), jnp.float32),
        ],
        compiler_params=pltpu.CompilerParams(needs_layout_passes=False),
    )
    def _lift(att_hbm, eidx_hbm, out_hbm, att_v, ei_v, eo_v, no_v):
        wid = lax.axis_index("s") * NC + lax.axis_index("c")
        pltpu.sync_copy(att_hbm, att_v)
        base = 128 * jnp.where(wid < N_BIG, wid * T_BIG,
                               N_BIG * T_BIG + (wid - N_BIG) * T_SML)

        def run(sz):
            pltpu.sync_copy(eidx_hbm.at[:, pl.ds(base, sz)],
                            ei_v.at[:, pl.ds(0, sz)])

            @plsc.parallel_loop(0, sz // L, 1, unroll=8)
            def _edge_body(i):
                s = plsc.load_gather(att_v, [ei_v[0, pl.ds(i * L, L)]])
                d = plsc.load_gather(att_v, [ei_v[1, pl.ds(i * L, L)]])
                eo_v[pl.ds(i * L, L)] = s * d

            pltpu.sync_copy(eo_v.at[pl.ds(0, sz)],
                            out_hbm.at[pl.ds(base, sz)])

        @pl.when(wid < N_BIG)
        def _big():
            run(SZ_BIG)

        @pl.when(wid >= N_BIG)
        def _small():
            run(SZ_SML)

        @pl.when(wid < N_WORKERS_NODE)
        def _node_part():
            nbase = wid * N_PER

            def node_body(i, carry):
                a = att_v[pl.ds(nbase + i * L, L)]
                no_v[pl.ds(i * L, L)] = a * a
                return carry

            lax.fori_loop(0, N_PER // L, node_body, 0)
            pltpu.sync_copy(no_v, out_hbm.at[pl.ds(E + nbase, N_PER)])

    return _lift


def kernel(x, edge_index, W1, b1, W2, b2, W3, b3, noise):
    rn = (jnp.log(noise) - jnp.log(1.0 - noise)).reshape(N)
    att = _mlp(x, W1, W2, W3.reshape(1, D), b3.reshape(1, 1), rn)
    out = _build_lift()(att, edge_index)
    return out.reshape(E + N, 1)


# speedup vs baseline: 113.8077x; 1.0339x over previous
import functools

import jax
import jax.numpy as jnp
from jax import lax
from jax.experimental import pallas as pl
from jax.experimental.pallas import tpu as pltpu
from jax.experimental.pallas import tpu_sc as plsc

N = 10000
E = 320000
D = 128

NC = 2
NS = 16
NW = NC * NS
L = 16

CT = E // 128
T_SML = CT // NW
T_BIG = T_SML + 1
N_BIG = CT - NW * T_SML
SZ_BIG = T_BIG * 128
SZ_SML = T_SML * 128

N_CHUNKS = N // L
N_WORKERS_NODE = 25
N_PER = (N_CHUNKS // N_WORKERS_NODE) * L


def _norm_relu(h):
    eps = 1e-5
    inv_n = 1.0 / N
    m = jnp.sum(h, axis=1, keepdims=True) * inv_n
    s2 = jnp.sum(h * h, axis=1, keepdims=True) * inv_n
    r = lax.rsqrt(s2 - m * m + eps)
    return jnp.maximum(h * r - m * r, 0.0)


def _mlp_body(x_ref, w1_ref, w2_ref, w3_ref, b3_ref, rn_ref, att_ref):
    h = lax.dot_general(w1_ref[...], x_ref[...], (((0,), (1,)), ((), ())),
                        preferred_element_type=jnp.float32)
    h = _norm_relu(h)
    h = lax.dot_general(w2_ref[...], h, (((0,), (0,)), ((), ())),
                        preferred_element_type=jnp.float32)
    h = _norm_relu(h)
    logit = lax.dot_general(w3_ref[...], h, (((1,), (0,)), ((), ())),
                            preferred_element_type=jnp.float32)
    logit = logit + b3_ref[...]
    z = logit + rn_ref[...].reshape(1, N)
    att_ref[...] = jax.nn.sigmoid(z).reshape(N)


_mlp = pl.pallas_call(
    _mlp_body,
    out_shape=jax.ShapeDtypeStruct((N,), jnp.float32),
)


@functools.cache
def _build_lift():
    mesh = plsc.VectorSubcoreMesh(core_axis_name="c", subcore_axis_name="s")

    @functools.partial(
        pl.kernel,
        mesh=mesh,
        out_type=jax.ShapeDtypeStruct((E + N,), jnp.float32),
        scratch_types=[
            pltpu.VMEM((N,), jnp.float32),
            pltpu.VMEM((2, SZ_BIG), jnp.int32),
            pltpu.VMEM((SZ_BIG,), jnp.float32),
            pltpu.VMEM((N_PER,), jnp.float32),
        ],
        compiler_params=pltpu.CompilerParams(needs_layout_passes=False),
    )
    def _lift(att_hbm, eidx_hbm, out_hbm, att_v, ei_v, eo_v, no_v):
        wid = lax.axis_index("s") * NC + lax.axis_index("c")
        pltpu.sync_copy(att_hbm, att_v)
        base = 128 * jnp.where(wid < N_BIG, wid * T_BIG,
                               N_BIG * T_BIG + (wid - N_BIG) * T_SML)

        def run(sz):
            pltpu.sync_copy(eidx_hbm.at[:, pl.ds(base, sz)],
                            ei_v.at[:, pl.ds(0, sz)])

            @plsc.parallel_loop(0, sz // L, 1, unroll=16)
            def _edge_body(i):
                s = plsc.load_gather(att_v, [ei_v[0, pl.ds(i * L, L)]])
                d = plsc.load_gather(att_v, [ei_v[1, pl.ds(i * L, L)]])
                eo_v[pl.ds(i * L, L)] = s * d

            pltpu.sync_copy(eo_v.at[pl.ds(0, sz)],
                            out_hbm.at[pl.ds(base, sz)])

        @pl.when(wid < N_BIG)
        def _big():
            run(SZ_BIG)

        @pl.when(wid >= N_BIG)
        def _small():
            run(SZ_SML)

        @pl.when(wid < N_WORKERS_NODE)
        def _node_part():
            nbase = wid * N_PER

            def node_body(i, carry):
                a = att_v[pl.ds(nbase + i * L, L)]
                no_v[pl.ds(i * L, L)] = a * a
                return carry

            lax.fori_loop(0, N_PER // L, node_body, 0)
            pltpu.sync_copy(no_v, out_hbm.at[pl.ds(E + nbase, N_PER)])

    return _lift


def kernel(x, edge_index, W1, b1, W2, b2, W3, b3, noise):
    rn = (jnp.log(noise) - jnp.log(1.0 - noise)).reshape(N)
    att = _mlp(x, W1, W2, W3.reshape(1, D), b3.reshape(1, 1), rn)
    out = _build_lift()(att, edge_index)
    return out.reshape(E + N, 1)


# speedup vs baseline: 115.7360x; 1.0169x over previous
import functools

import jax
import jax.numpy as jnp
from jax import lax
from jax.experimental import pallas as pl
from jax.experimental.pallas import tpu as pltpu
from jax.experimental.pallas import tpu_sc as plsc

N = 10000
E = 320000
D = 128

NC = 2
NS = 16
NW = NC * NS
L = 16

CT = E // 128
T_SML = CT // NW
T_BIG = T_SML + 1
N_BIG = CT - NW * T_SML
SZ_BIG = T_BIG * 128
SZ_SML = T_SML * 128

N_CHUNKS = N // L
N_WORKERS_NODE = 25
N_PER = (N_CHUNKS // N_WORKERS_NODE) * L


def _norm_relu(h):
    eps = 1e-5
    inv_n = 1.0 / N
    m = jnp.sum(h, axis=1, keepdims=True) * inv_n
    s2 = jnp.sum(h * h, axis=1, keepdims=True) * inv_n
    r = lax.rsqrt(s2 - m * m + eps)
    return jnp.maximum(h * r - m * r, 0.0)


def _mlp_body(x_ref, w1_ref, w2_ref, w3_ref, b3_ref, rn_ref, att_ref):
    h = lax.dot_general(w1_ref[...], x_ref[...], (((0,), (1,)), ((), ())),
                        preferred_element_type=jnp.float32)
    h = _norm_relu(h)
    h = lax.dot_general(w2_ref[...], h, (((0,), (0,)), ((), ())),
                        preferred_element_type=jnp.float32)
    h = _norm_relu(h)
    logit = lax.dot_general(w3_ref[...], h, (((1,), (0,)), ((), ())),
                            preferred_element_type=jnp.float32)
    logit = logit + b3_ref[...]
    z = logit + rn_ref[...].reshape(1, N)
    att_ref[...] = jax.nn.sigmoid(z).reshape(N)


_mlp = pl.pallas_call(
    _mlp_body,
    out_shape=jax.ShapeDtypeStruct((N,), jnp.float32),
)


@functools.cache
def _build_lift():
    mesh = plsc.VectorSubcoreMesh(core_axis_name="c", subcore_axis_name="s")

    @functools.partial(
        pl.kernel,
        mesh=mesh,
        out_type=jax.ShapeDtypeStruct((E + N,), jnp.float32),
        scratch_types=[
            pltpu.VMEM((N,), jnp.float32),
            pltpu.VMEM((2, SZ_BIG), jnp.int32),
            pltpu.VMEM((SZ_BIG,), jnp.float32),
            pltpu.VMEM((N_PER,), jnp.float32),
            pltpu.SemaphoreType.DMA,
            pltpu.SemaphoreType.DMA,
            pltpu.SemaphoreType.DMA,
            pltpu.SemaphoreType.DMA,
        ],
        compiler_params=pltpu.CompilerParams(needs_layout_passes=False),
    )
    def _lift(att_hbm, eidx_hbm, out_hbm, att_v, ei_v, eo_v, no_v,
              sem_att, sem_i0, sem_i1, sem_out):
        wid = lax.axis_index("s") * NC + lax.axis_index("c")
        base = 128 * jnp.where(wid < N_BIG, wid * T_BIG,
                               N_BIG * T_BIG + (wid - N_BIG) * T_SML)

        def run(sz):
            h0 = (sz // 2) // 128 * 128
            h1 = sz - h0
            c_att = pltpu.async_copy(att_hbm, att_v, sem_att)
            c_i0 = pltpu.async_copy(eidx_hbm.at[:, pl.ds(base, h0)],
                                    ei_v.at[:, pl.ds(0, h0)], sem_i0)
            c_i1 = pltpu.async_copy(eidx_hbm.at[:, pl.ds(base + h0, h1)],
                                    ei_v.at[:, pl.ds(h0, h1)], sem_i1)

            def gather_span(lo, hi):
                @plsc.parallel_loop(lo, hi, 1, unroll=16)
                def _edge_body(i):
                    s = plsc.load_gather(att_v, [ei_v[0, pl.ds(i * L, L)]])
                    d = plsc.load_gather(att_v, [ei_v[1, pl.ds(i * L, L)]])
                    eo_v[pl.ds(i * L, L)] = s * d

            c_att.wait()
            c_i0.wait()
            gather_span(0, h0 // L)
            c_o0 = pltpu.async_copy(eo_v.at[pl.ds(0, h0)],
                                    out_hbm.at[pl.ds(base, h0)], sem_out)
            c_i1.wait()
            gather_span(h0 // L, sz // L)
            c_o1 = pltpu.async_copy(eo_v.at[pl.ds(h0, h1)],
                                    out_hbm.at[pl.ds(base + h0, h1)], sem_out)

            @pl.when(wid < N_WORKERS_NODE)
            def _node_part():
                nbase = wid * N_PER

                def node_body(i, carry):
                    a = att_v[pl.ds(nbase + i * L, L)]
                    no_v[pl.ds(i * L, L)] = a * a
                    return carry

                lax.fori_loop(0, N_PER // L, node_body, 0)
                pltpu.sync_copy(no_v, out_hbm.at[pl.ds(E + nbase, N_PER)])

            c_o0.wait()
            c_o1.wait()

        @pl.when(wid < N_BIG)
        def _big():
            run(SZ_BIG)

        @pl.when(wid >= N_BIG)
        def _small():
            run(SZ_SML)

    return _lift


def kernel(x, edge_index, W1, b1, W2, b2, W3, b3, noise):
    rn = (jnp.log(noise) - jnp.log(1.0 - noise)).reshape(N)
    att = _mlp(x, W1, W2, W3.reshape(1, D), b3.reshape(1, 1), rn)
    out = _build_lift()(att, edge_index)
    return out.reshape(E + N, 1)
